# baseline (speedup 1.0000x reference)
import dataclasses

import jax
import jax.numpy as jnp
from jax import lax
from jax.experimental import pallas as pl
from jax.experimental.pallas import tpu as pltpu
from jax.experimental.pallas import tpu_sc as plsc

M_ITEM = 1000000
DIM = 16
N_ROWS = 819200
LR = 0.01
NC = 2
NS = 16
NW = NC * NS
R = 1000
RREG = 1008
NCHUNK = M_ITEM // R
GB = 512
QN = GB // 128
OFFS_PAD = 1024


def _body(gt_hbm, items_hbm, offs_hbm, out_hbm,
          shacc, shcnt, accv, cntv, updv, gvt, gv, ids, idxs, ones, offs):
    cidx = lax.axis_index("c")
    s = lax.axis_index("s")
    w = s * NC + cidx
    base = pl.multiple_of(s * RREG, 8)
    pltpu.sync_copy(offs_hbm, offs)

    zero16 = jnp.zeros((DIM,), jnp.float32)
    one16 = jnp.ones((DIM,), jnp.float32)

    @pl.loop(0, RREG)
    def _z(r):
        accv[r] = zero16
        cntv[r] = zero16

    @pl.loop(0, 128)
    def _o(r):
        ones[r] = one16

    pltpu.sync_copy(accv, shacc.at[pl.ds(base, RREG), :])
    pltpu.sync_copy(cntv, shcnt.at[pl.ds(base, RREG), :])

    kmax = (jnp.int32(NCHUNK) - w + (NW - 1)) // NW
    iota16 = lax.iota(jnp.int32, DIM)

    def _chunk(k, carry0):
        c = w + k * NW
        tb = pl.multiple_of(c * R, 8)
        ovec = offs[pl.ds(c, 16)]
        lo = ovec[0]
        hi = ovec[1]
        lo8 = lo & jnp.int32(-8)
        nb = (hi - lo8 + (GB - 1)) // GB

        def batch(b, carry):
            s0 = lo8 + b * GB
            st = jnp.minimum(s0, jnp.int32(N_ROWS - GB))
            st = pl.multiple_of(st, 8)
            d = s0 - st
            pltpu.sync_copy(gt_hbm.at[:, pl.ds(st, GB)], gvt)
            pltpu.sync_copy(items_hbm.at[pl.ds(st, GB)], ids)

            def clipgrp(jb, cc):
                j0 = jb * DIM
                cols = [gvt.at[t][pl.ds(j0, DIM)] for t in range(DIM)]
                ssv = cols[0] * cols[0]
                for t in range(1, DIM):
                    ssv = ssv + cols[t] * cols[t]
                bits = lax.bitcast_convert_type(ssv, jnp.int32)
                y = lax.bitcast_convert_type(
                    jnp.int32(0x5F3759DF) - (bits >> 1), jnp.float32)
                y = y * (1.5 - 0.5 * ssv * y * y)
                y = y * (1.5 - 0.5 * ssv * y * y)
                y = y * (1.5 - 0.5 * ssv * y * y)
                scale = jnp.where(ssv > 1.0, y, 1.0)
                rowidx = iota16 + j0
                for t in range(DIM):
                    colidx = jnp.full((DIM,), t, jnp.int32)
                    plsc.store_scatter(gv, [rowidx, colidx],
                                       cols[t] * scale)
                return cc

            lax.fori_loop(0, GB // DIM, clipgrp, 0)

            for q in range(QN):
                def idxgrp(jb, cc, q=q):
                    p0 = q * 128 + jb * DIM
                    idvec = ids[pl.ds(p0, DIM)]
                    i1 = idvec - tb
                    i2 = jnp.minimum(jnp.maximum(i1, -1), R)
                    i2 = jnp.where(i2 < 0, R, i2)
                    i2 = jnp.where(iota16 + p0 < d, R, i2)
                    idxs.at[q][pl.ds(jb * DIM, DIM)] = i2 + base
                    return cc

                lax.fori_loop(0, 128 // DIM, idxgrp, 0)
                pltpu.sync_copy(gv.at[pl.ds(q * 128, 128), :],
                                shacc.at[idxs.at[q]], add=True)
                pltpu.sync_copy(ones, shcnt.at[idxs.at[q]], add=True)
            return carry

        lax.fori_loop(0, nb, batch, 0)

        pltpu.sync_copy(shacc.at[pl.ds(base, R), :], accv.at[pl.ds(0, R), :])
        pltpu.sync_copy(shcnt.at[pl.ds(base, R), :], cntv.at[pl.ds(0, R), :])

        def comb(r, carry):
            a = accv[r]
            cl = cntv[r]
            u = LR * (a / jnp.maximum(cl, 1.0))
            plsc.store_scatter(updv, [iota16, jnp.full((DIM,), r, jnp.int32)],
                               u)
            accv[r] = zero16
            cntv[r] = zero16
            return carry

        lax.fori_loop(0, R, comb, 0)
        pltpu.sync_copy(accv, shacc.at[pl.ds(base, RREG), :])
        pltpu.sync_copy(cntv, shcnt.at[pl.ds(base, RREG), :])
        pltpu.sync_copy(updv, out_hbm.at[:, pl.ds(tb, R)])
        return carry0

    lax.fori_loop(0, kmax, _chunk, 0)


def kernel(items_emb, items_emb_grad, items):
    boundaries = jnp.arange(NCHUNK + 1, dtype=jnp.int32) * R
    offs = jnp.searchsorted(items, boundaries, side="left").astype(jnp.int32)
    offs = jnp.pad(offs, (0, OFFS_PAD - (NCHUNK + 1)),
                   constant_values=N_ROWS)

    mesh = plsc.VectorSubcoreMesh(core_axis_name="c", subcore_axis_name="s")
    cp = pltpu.CompilerParams()
    if "needs_layout_passes" in pltpu.CompilerParams.__dataclass_fields__:
        cp = dataclasses.replace(cp, needs_layout_passes=False)
    if "use_tc_tiling_on_sc" in pltpu.CompilerParams.__dataclass_fields__:
        cp = dataclasses.replace(cp, use_tc_tiling_on_sc=False)
    run = pl.kernel(
        _body,
        compiler_params=cp,
        out_type=jax.ShapeDtypeStruct((DIM, M_ITEM), jnp.float32),
        mesh=mesh,
        scratch_types=[
            pltpu.VMEM_SHARED((NS * RREG, DIM), jnp.float32),
            pltpu.VMEM_SHARED((NS * RREG, DIM), jnp.float32),
            pltpu.VMEM((RREG, DIM), jnp.float32),
            pltpu.VMEM((RREG, DIM), jnp.float32),
            pltpu.VMEM((DIM, R), jnp.float32),
            pltpu.VMEM((DIM, GB), jnp.float32),
            pltpu.VMEM((GB, DIM), jnp.float32),
            pltpu.VMEM((GB,), jnp.int32),
            pltpu.VMEM((QN, 128), jnp.int32),
            pltpu.VMEM((128, DIM), jnp.float32),
            pltpu.VMEM((OFFS_PAD,), jnp.int32),
        ],
    )
    upd = run(items_emb_grad.T, items, offs)
    return (items_emb.T - upd).T


# speedup vs baseline: 2.1719x; 2.1719x over previous
import dataclasses

import jax
import jax.numpy as jnp
from jax import lax
from jax.experimental import pallas as pl
from jax.experimental.pallas import tpu as pltpu
from jax.experimental.pallas import tpu_sc as plsc

M_ITEM = 1000000
DIM = 16
N_ROWS = 819200
LR = 0.01
NC = 2
NS = 16
NW = NC * NS
R = 1000
RREG = 1008
NCHUNK = M_ITEM // R
GB = 512
QN = GB // 128
OFFS_PAD = 1024


def _body(gt_hbm, items_hbm, offs_hbm, out_hbm,
          shacc, shcnt, accv, cntv, updv, gvt, gv, ids, idxs, ones, offs):
    cidx = lax.axis_index("c")
    s = lax.axis_index("s")
    w = s * NC + cidx
    base = pl.multiple_of(s * RREG, 8)
    pltpu.sync_copy(offs_hbm, offs)

    zero16 = jnp.zeros((DIM,), jnp.float32)
    one16 = jnp.ones((DIM,), jnp.float32)

    @pl.loop(0, RREG)
    def _z(r):
        accv[r] = zero16
        cntv[r] = zero16

    @pl.loop(0, 128)
    def _o(r):
        ones[r] = one16

    pltpu.sync_copy(accv, shacc.at[pl.ds(base, RREG), :])
    pltpu.sync_copy(cntv, shcnt.at[pl.ds(base, RREG), :])

    kmax = (jnp.int32(NCHUNK) - w + (NW - 1)) // NW
    iota16 = lax.iota(jnp.int32, DIM)

    def _chunk(k, carry0):
        c = w + k * NW
        tb = pl.multiple_of(c * R, 8)
        ovec = offs[pl.ds(c, 16)]
        lo = ovec[0]
        hi = ovec[1]
        lo8 = lo & jnp.int32(-8)
        nb = (hi - lo8 + (GB - 1)) // GB

        def batch(b, carry):
            s0 = lo8 + b * GB
            st = jnp.minimum(s0, jnp.int32(N_ROWS - GB))
            st = pl.multiple_of(st, 8)
            d = s0 - st
            pltpu.sync_copy(gt_hbm.at[:, pl.ds(st, GB)], gvt)
            pltpu.sync_copy(items_hbm.at[pl.ds(st, GB)], ids)

            def clipgrp(jb, cc):
                j0 = jb * DIM
                cols = [gvt.at[t][pl.ds(j0, DIM)] for t in range(DIM)]
                ssv = cols[0] * cols[0]
                for t in range(1, DIM):
                    ssv = ssv + cols[t] * cols[t]
                bits = lax.bitcast_convert_type(ssv, jnp.int32)
                y = lax.bitcast_convert_type(
                    jnp.int32(0x5F3759DF) - (bits >> 1), jnp.float32)
                y = y * (1.5 - 0.5 * ssv * y * y)
                y = y * (1.5 - 0.5 * ssv * y * y)
                y = y * (1.5 - 0.5 * ssv * y * y)
                scale = jnp.where(ssv > 1.0, y, 1.0)
                rowidx = iota16 + j0
                for t in range(DIM):
                    colidx = jnp.full((DIM,), t, jnp.int32)
                    plsc.store_scatter(gv, [rowidx, colidx],
                                       cols[t] * scale)
                return cc

            lax.fori_loop(0, GB // DIM, clipgrp, 0)

            for q in range(QN):
                def idxgrp(jb, cc, q=q):
                    p0 = q * 128 + jb * DIM
                    idvec = ids[pl.ds(p0, DIM)]
                    i1 = idvec - tb
                    i2 = jnp.minimum(jnp.maximum(i1, -1), R)
                    i2 = jnp.where(i2 < 0, R, i2)
                    i2 = jnp.where(iota16 + p0 < d, R, i2)
                    idxs.at[q][pl.ds(jb * DIM, DIM)] = i2 + base
                    return cc

                lax.fori_loop(0, 128 // DIM, idxgrp, 0)
                pltpu.sync_copy(gv.at[pl.ds(q * 128, 128), :],
                                shacc.at[idxs.at[q]], add=True)
                pltpu.sync_copy(ones, shcnt.at[idxs.at[q]], add=True)
            return carry

        lax.fori_loop(0, nb, batch, 0)

        pltpu.sync_copy(shacc.at[pl.ds(base, R), :], accv.at[pl.ds(0, R), :])
        pltpu.sync_copy(shcnt.at[pl.ds(base, R), :], cntv.at[pl.ds(0, R), :])

        def comb(r, carry):
            a = accv[r]
            cl = cntv[r]
            updv[pl.ds(r * DIM, DIM)] = LR * (a / jnp.maximum(cl, 1.0))
            accv[r] = zero16
            cntv[r] = zero16
            return carry

        lax.fori_loop(0, R, comb, 0)
        pltpu.sync_copy(accv, shacc.at[pl.ds(base, RREG), :])
        pltpu.sync_copy(cntv, shcnt.at[pl.ds(base, RREG), :])
        pltpu.sync_copy(updv, out_hbm.at[pl.ds(tb * DIM, R * DIM)])
        return carry0

    lax.fori_loop(0, kmax, _chunk, 0)


def kernel(items_emb, items_emb_grad, items):
    boundaries = jnp.arange(NCHUNK + 1, dtype=jnp.int32) * R
    offs = jnp.searchsorted(items, boundaries, side="left").astype(jnp.int32)
    offs = jnp.pad(offs, (0, OFFS_PAD - (NCHUNK + 1)),
                   constant_values=N_ROWS)

    mesh = plsc.VectorSubcoreMesh(core_axis_name="c", subcore_axis_name="s")
    cp = pltpu.CompilerParams()
    if "needs_layout_passes" in pltpu.CompilerParams.__dataclass_fields__:
        cp = dataclasses.replace(cp, needs_layout_passes=False)
    if "use_tc_tiling_on_sc" in pltpu.CompilerParams.__dataclass_fields__:
        cp = dataclasses.replace(cp, use_tc_tiling_on_sc=False)
    run = pl.kernel(
        _body,
        compiler_params=cp,
        out_type=jax.ShapeDtypeStruct((M_ITEM * DIM,), jnp.float32),
        mesh=mesh,
        scratch_types=[
            pltpu.VMEM_SHARED((NS * RREG, DIM), jnp.float32),
            pltpu.VMEM_SHARED((NS * RREG, DIM), jnp.float32),
            pltpu.VMEM((RREG, DIM), jnp.float32),
            pltpu.VMEM((RREG, DIM), jnp.float32),
            pltpu.VMEM((R * DIM,), jnp.float32),
            pltpu.VMEM((DIM, GB), jnp.float32),
            pltpu.VMEM((GB, DIM), jnp.float32),
            pltpu.VMEM((GB,), jnp.int32),
            pltpu.VMEM((QN, 128), jnp.int32),
            pltpu.VMEM((128, DIM), jnp.float32),
            pltpu.VMEM((OFFS_PAD,), jnp.int32),
        ],
    )
    upd = run(items_emb_grad.T, items, offs)
    return items_emb - upd.reshape(M_ITEM, DIM)
